# baseline (speedup 1.0000x reference)
import functools

import jax
import jax.numpy as jnp
from jax import lax
from jax.experimental import pallas as pl
from jax.experimental.pallas import tpu as pltpu
from jax.experimental.pallas import tpu_sc as plsc

NUM_EMB = 8192
DIM = 128
B = 64
HW = 1024

NC = 2
NS = 16
L = 16

D_PER_TILE = DIM // (NS * 8) * 8
B_PER_CORE = B // NC


def _body(q_hbm, w_hbm, out_hbm, colbuf, idxbuf, outbuf):
    c = lax.axis_index("c")
    s = lax.axis_index("s")

    pltpu.sync_copy(w_hbm.at[:, pl.ds(s * 8, 8)], colbuf)

    def per_image(j, carry):
        b = c * B_PER_CORE + j
        pltpu.sync_copy(q_hbm.at[pl.ds(b * HW, HW)], idxbuf)

        def per_vec(i, carry2):
            qv = idxbuf[pl.ds(i * L, L)]
            for d in range(8):
                dv = jnp.full((L,), d, jnp.int32)
                val = plsc.load_gather(colbuf, [qv, dv])
                outbuf[pl.ds(d * HW + i * L, L)] = val
            return carry2

        lax.fori_loop(0, HW // L, per_vec, 0, unroll=4)
        pltpu.sync_copy(outbuf, out_hbm.at[b, pl.ds(s * 8 * HW, 8 * HW)])
        return carry

    lax.fori_loop(0, B_PER_CORE, per_image, 0)


@jax.jit
def _lookup(q_flat, w):
    mesh = plsc.VectorSubcoreMesh(core_axis_name="c", subcore_axis_name="s")
    f = pl.kernel(
        _body,
        out_type=jax.ShapeDtypeStruct((B, DIM * HW), jnp.float32),
        mesh=mesh,
        scratch_types=[
            pltpu.VMEM((NUM_EMB, 8), jnp.float32),
            pltpu.VMEM((HW,), jnp.int32),
            pltpu.VMEM((8 * HW,), jnp.float32),
        ],
        compiler_params=pltpu.CompilerParams(
            use_tc_tiling_on_sc=False, needs_layout_passes=False
        ),
    )
    return f(q_flat, w)


def kernel(quantized, embedding_weight):
    q_flat = quantized.reshape(B * HW)
    out = _lookup(q_flat, embedding_weight)
    return (quantized, out.reshape(B, DIM, 32, 32))


# speedup vs baseline: 1.7635x; 1.7635x over previous
import jax
import jax.numpy as jnp
from jax import lax
from jax.experimental import pallas as pl
from jax.experimental.pallas import tpu as pltpu
from jax.experimental.pallas import tpu_sc as plsc

NUM_EMB = 8192
DIM = 128
B = 64
HW = 1024

NC = 2
NS = 16
L = 16

D_PER_TILE = DIM // (NC * NS) * NC
B_PER_CORE = B // NC
OUT_W = D_PER_TILE * HW


def _body(q_hbm, w_hbm, out_hbm, colbuf, idxbuf, outbuf, sem0, sem1):
    c = lax.axis_index("c")
    s = lax.axis_index("s")

    pltpu.sync_copy(w_hbm.at[:, pl.ds(s * D_PER_TILE, D_PER_TILE)], colbuf)
    pltpu.sync_copy(q_hbm.at[pl.ds(c * B_PER_CORE * HW, B_PER_CORE * HW)], idxbuf)

    sems = (sem0, sem1)

    def gather_image(j, p):
        @plsc.parallel_loop(0, HW // L, unroll=4)
        def _(i):
            qv = idxbuf[pl.ds(j * HW + i * L, L)]
            for d in range(D_PER_TILE):
                dv = jnp.full((L,), d, jnp.int32)
                val = plsc.load_gather(colbuf, [qv, dv])
                outbuf[pl.ds(p * OUT_W + d * HW + i * L, L)] = val

    def out_slice(j):
        b = c * B_PER_CORE + j
        return out_hbm.at[b, pl.ds(s * OUT_W, OUT_W)]

    def start_store(j, p):
        src = outbuf.at[pl.ds(p * OUT_W, OUT_W)]
        pltpu.async_copy(src, out_slice(j), sems[p])

    def wait_store(j, p):
        src = outbuf.at[pl.ds(p * OUT_W, OUT_W)]
        pltpu.make_async_copy(src, out_slice(j), sems[p]).wait()

    def per_pair(jj, carry):
        for p in range(2):
            j = 2 * jj + p

            @pl.when(jj > 0)
            def _():
                wait_store(j - 2, p)

            gather_image(j, p)
            start_store(j, p)
        return carry

    lax.fori_loop(0, B_PER_CORE // 2, per_pair, 0)
    wait_store(B_PER_CORE - 2, 0)
    wait_store(B_PER_CORE - 1, 1)


@jax.jit
def _lookup(q_flat, w):
    mesh = plsc.VectorSubcoreMesh(core_axis_name="c", subcore_axis_name="s")
    f = pl.kernel(
        _body,
        out_type=jax.ShapeDtypeStruct((B, DIM * HW), jnp.float32),
        mesh=mesh,
        scratch_types=[
            pltpu.VMEM((NUM_EMB, D_PER_TILE), jnp.float32),
            pltpu.VMEM((B_PER_CORE * HW,), jnp.int32),
            pltpu.VMEM((2 * OUT_W,), jnp.float32),
            pltpu.SemaphoreType.DMA,
            pltpu.SemaphoreType.DMA,
        ],
        compiler_params=pltpu.CompilerParams(
            use_tc_tiling_on_sc=False, needs_layout_passes=False
        ),
    )
    return f(q_flat, w)


def kernel(quantized, embedding_weight):
    q_flat = quantized.reshape(B * HW)
    out = _lookup(q_flat, embedding_weight)
    return (quantized, out.reshape(B, DIM, 32, 32))


# speedup vs baseline: 5.5727x; 3.1600x over previous
import jax
import jax.numpy as jnp
from jax import lax
from jax.experimental import pallas as pl
from jax.experimental.pallas import tpu as pltpu
from jax.experimental.pallas import tpu_sc as plsc

NUM_EMB = 8192
DIM = 128
B = 64
HW = 1024
N = B * HW

NC = 2
NS = 16
NW = NC * NS

PER_TILE = N // NW
CHUNK = 128
NCHUNK = PER_TILE // CHUNK
NBUF = 4


def _body(q_hbm, w_hbm, out_hbm, idxbuf, rows, gsems, ssems):
    c = lax.axis_index("c")
    s = lax.axis_index("s")
    wid = s * NC + c
    base = wid * PER_TILE

    pltpu.sync_copy(q_hbm.at[pl.ds(base, PER_TILE)], idxbuf)

    def start_gather(ch, p):
        idx = idxbuf.at[pl.ds(ch * CHUNK, CHUNK)]
        pltpu.async_copy(w_hbm.at[idx], rows.at[p], gsems.at[p])

    def wait_gather(ch, p):
        idx = idxbuf.at[pl.ds(ch * CHUNK, CHUNK)]
        pltpu.make_async_copy(w_hbm.at[idx], rows.at[p], gsems.at[p]).wait()

    def out_slice(ch):
        return out_hbm.at[pl.ds(base + ch * CHUNK, CHUNK), :]

    def start_store(ch, p):
        pltpu.async_copy(rows.at[p], out_slice(ch), ssems.at[p])

    def wait_store(ch, p):
        pltpu.make_async_copy(rows.at[p], out_slice(ch), ssems.at[p]).wait()

    for p in range(NBUF):
        start_gather(p, p)

    def per_round(cc, carry):
        for p in range(NBUF):
            ch = NBUF * cc + p
            wait_gather(ch, p)
            start_store(ch, p)

            @pl.when(cc < NCHUNK // NBUF - 1)
            def _():
                wait_store(ch, p)
                start_gather(ch + NBUF, p)

        return carry

    lax.fori_loop(0, NCHUNK // NBUF, per_round, 0)
    for p in range(NBUF):
        wait_store(NCHUNK - NBUF + p, p)


@jax.jit
def _lookup(q_flat, w):
    mesh = plsc.VectorSubcoreMesh(core_axis_name="c", subcore_axis_name="s")
    f = pl.kernel(
        _body,
        out_type=jax.ShapeDtypeStruct((N, DIM), jnp.float32),
        mesh=mesh,
        scratch_types=[
            pltpu.VMEM((PER_TILE,), jnp.int32),
            pltpu.VMEM((NBUF, CHUNK, DIM), jnp.float32),
            pltpu.SemaphoreType.DMA((NBUF,)),
            pltpu.SemaphoreType.DMA((NBUF,)),
        ],
        compiler_params=pltpu.CompilerParams(
            use_tc_tiling_on_sc=False, needs_layout_passes=False
        ),
    )
    return f(q_flat, w)


def kernel(quantized, embedding_weight):
    q_flat = quantized.reshape(N)
    rows = _lookup(q_flat, embedding_weight)
    emb = rows.reshape(B, 32, 32, DIM).transpose(0, 3, 1, 2)
    return (quantized, emb)


# speedup vs baseline: 5.6384x; 1.0118x over previous
import jax
import jax.numpy as jnp
from jax import lax
from jax.experimental import pallas as pl
from jax.experimental.pallas import tpu as pltpu
from jax.experimental.pallas import tpu_sc as plsc

NUM_EMB = 8192
DIM = 128
B = 64
HW = 1024
N = B * HW

NC = 2
NS = 16
NW = NC * NS

PER_TILE = N // NW
CHUNK = 128
NCHUNK = PER_TILE // CHUNK
NBUF = 6


def _body(q_hbm, w_hbm, out_hbm, idxbuf, rows, gsems, ssems):
    c = lax.axis_index("c")
    s = lax.axis_index("s")
    wid = s * NC + c
    base = wid * PER_TILE

    pltpu.sync_copy(q_hbm.at[pl.ds(base, PER_TILE)], idxbuf)

    def start_gather(ch, p):
        idx = idxbuf.at[pl.ds(ch * CHUNK, CHUNK)]
        pltpu.async_copy(w_hbm.at[idx], rows.at[p], gsems.at[p])

    def wait_gather(ch, p):
        idx = idxbuf.at[pl.ds(ch * CHUNK, CHUNK)]
        pltpu.make_async_copy(w_hbm.at[idx], rows.at[p], gsems.at[p]).wait()

    def out_slice(ch):
        return out_hbm.at[pl.ds(base + ch * CHUNK, CHUNK), :]

    def start_store(ch, p):
        pltpu.async_copy(rows.at[p], out_slice(ch), ssems.at[p])

    def wait_store(ch, p):
        pltpu.make_async_copy(rows.at[p], out_slice(ch), ssems.at[p]).wait()

    for ch in range(min(NBUF, NCHUNK)):
        start_gather(ch, ch % NBUF)

    for ch in range(NCHUNK):
        p = ch % NBUF
        wait_gather(ch, p)
        start_store(ch, p)
        if ch + NBUF < NCHUNK:
            wait_store(ch, p)
            start_gather(ch + NBUF, p)

    for ch in range(max(NCHUNK - NBUF, 0), NCHUNK):
        wait_store(ch, ch % NBUF)


@jax.jit
def _lookup(q_flat, w):
    mesh = plsc.VectorSubcoreMesh(core_axis_name="c", subcore_axis_name="s")
    f = pl.kernel(
        _body,
        out_type=jax.ShapeDtypeStruct((N, DIM), jnp.float32),
        mesh=mesh,
        scratch_types=[
            pltpu.VMEM((PER_TILE,), jnp.int32),
            pltpu.VMEM((NBUF, CHUNK, DIM), jnp.float32),
            pltpu.SemaphoreType.DMA((NBUF,)),
            pltpu.SemaphoreType.DMA((NBUF,)),
        ],
        compiler_params=pltpu.CompilerParams(
            use_tc_tiling_on_sc=False, needs_layout_passes=False
        ),
    )
    return f(q_flat, w)


def kernel(quantized, embedding_weight):
    q_flat = quantized.reshape(N)
    rows = _lookup(q_flat, embedding_weight)
    emb = rows.reshape(B, 32, 32, DIM).transpose(0, 3, 1, 2)
    return (quantized, emb)
